# baseline (speedup 1.0000x reference)
import functools

import numpy as np
import jax
import jax.numpy as jnp
from jax import lax
from jax.experimental import pallas as pl
from jax.experimental.pallas import tpu as pltpu
from jax.experimental.pallas import tpu_sc as plsc

N = 10000
E = 160000
H = 32
HH = H * H
N_LAYERS = 3

NC, NS = 2, 16
NW = NC * NS
CH = 128
CPW = 40
NB = 4
NGROUPS = CPW // NB
NCHUNK_P = NW * CPW
E_PAD = NCHUNK_P * CH

BE = 1280
BN = 2000
N_ACC = 12000
NSTRIPE = N_ACC // NS
NBLK_REAL = E // BE

_f32 = jnp.float32


def _dot(a, b):
    return jnp.dot(a, b, preferred_element_type=_f32)


def _embed_node_body(x_ref, w_ref, b_ref, o_ref):
    o_ref[...] = _dot(x_ref[...], w_ref[...]) + b_ref[...]


def _embed_node(x, wT, b):
    grid = N // BN
    return pl.pallas_call(
        _embed_node_body,
        grid=(grid,),
        in_specs=[
            pl.BlockSpec((BN, x.shape[1]), lambda i: (i, 0)),
            pl.BlockSpec(wT.shape, lambda i: (0, 0)),
            pl.BlockSpec(b.shape, lambda i: (0, 0)),
        ],
        out_specs=pl.BlockSpec((BN, H), lambda i: (i, 0)),
        out_shape=jax.ShapeDtypeStruct((N, H), _f32),
    )(x, wT, b)


def _edge_feat_body(x_ref, w_ref, b_ref, w1_ref, b1_ref, o_ref):
    t = _dot(x_ref[...], w_ref[...]) + b_ref[...]
    o_ref[...] = jnp.maximum(_dot(t, w1_ref[...]) + b1_ref[...], 0.0)


def _edge_feat(x, wT, b, w1T, b1):
    grid = E_PAD // BE
    return pl.pallas_call(
        _edge_feat_body,
        grid=(grid,),
        in_specs=[
            pl.BlockSpec((BE, x.shape[1]),
                         lambda i: (jnp.minimum(i, NBLK_REAL - 1), 0)),
            pl.BlockSpec(wT.shape, lambda i: (0, 0)),
            pl.BlockSpec(b.shape, lambda i: (0, 0)),
            pl.BlockSpec(w1T.shape, lambda i: (0, 0)),
            pl.BlockSpec(b1.shape, lambda i: (0, 0)),
        ],
        out_specs=pl.BlockSpec((BE, H), lambda i: (i, 0)),
        out_shape=jax.ShapeDtypeStruct((E_PAD, H), _f32),
    )(x, wT, b, w1T, b1)


def _msg_body(hs_ref, ef_ref, w2_ref, b2c_ref, o_ref):
    @pl.when(pl.program_id(0) >= NBLK_REAL)
    def _():
        o_ref[...] = jnp.zeros_like(o_ref)

    @pl.when(pl.program_id(0) < NBLK_REAL)
    def _():
        gT = lax.dot_general(w2_ref[...], ef_ref[...], (((1,), (1,)), ((), ())),
                             preferred_element_type=_f32) + b2c_ref[...]
        hsT = hs_ref[...].T
        acc = gT[0:H, :] * hsT[0:1, :]
        for i in range(1, H):
            acc += gT[i * H:(i + 1) * H, :] * hsT[i:i + 1, :]
        o_ref[...] = acc.T


def _msg(hs, ef, w2, b2c):
    grid = E_PAD // BE
    return pl.pallas_call(
        _msg_body,
        grid=(grid,),
        in_specs=[
            pl.BlockSpec((BE, H), lambda i: (i, 0)),
            pl.BlockSpec((BE, H), lambda i: (i, 0)),
            pl.BlockSpec(w2.shape, lambda i: (0, 0)),
            pl.BlockSpec(b2c.shape, lambda i: (0, 0)),
        ],
        out_specs=pl.BlockSpec((BE, H), lambda i: (i, 0)),
        out_shape=jax.ShapeDtypeStruct((E_PAD, H), _f32),
    )(hs, ef, w2, b2c)


def _gru_body(p0_ref, p1_ref, d0_ref, d1_ref, h_ref,
              wir_ref, wiz_ref, win_ref, whr_ref, whz_ref, whn_ref,
              br_ref, bz_ref, bin_ref, bhn_ref, o_ref):
    deg = jnp.maximum(d0_ref[...] + d1_ref[...], 1.0)
    m = jnp.maximum((p0_ref[...] + p1_ref[...]) / deg, 0.0)
    h = h_ref[...]
    r = jax.nn.sigmoid(_dot(m, wir_ref[...]) + _dot(h, whr_ref[...]) + br_ref[...])
    z = jax.nn.sigmoid(_dot(m, wiz_ref[...]) + _dot(h, whz_ref[...]) + bz_ref[...])
    n = jnp.tanh(_dot(m, win_ref[...]) + bin_ref[...]
                 + r * (_dot(h, whn_ref[...]) + bhn_ref[...]))
    o_ref[...] = (1.0 - z) * n + z * h


def _gru(mp, degp, h, mats, biases):
    grid = N // BN
    off = N_ACC // BN
    blk = pl.BlockSpec((BN, H), lambda i: (i, 0))
    blk1 = pl.BlockSpec((BN, H), lambda i: (i + off, 0))
    full = lambda a: pl.BlockSpec(a.shape, lambda i: (0, 0))
    return pl.pallas_call(
        _gru_body,
        grid=(grid,),
        in_specs=[blk, blk1, blk, blk1, blk]
        + [full(m) for m in mats] + [full(b) for b in biases],
        out_specs=blk,
        out_shape=jax.ShapeDtypeStruct((N, H), _f32),
    )(mp, mp, degp, degp, h, *mats, *biases)


@functools.cache
def _build_sc_kernels():
    mesh = plsc.VectorSubcoreMesh(core_axis_name="c", subcore_axis_name="s",
                                  num_cores=NC, num_subcores=NS)

    @functools.partial(
        pl.kernel, mesh=mesh,
        out_type=jax.ShapeDtypeStruct((E_PAD, H), _f32),
        scratch_types=[
            pltpu.VMEM((CPW, CH), jnp.int32),
            pltpu.VMEM((NB, CH, H), _f32),
            pltpu.SemaphoreType.DMA,
            pltpu.SemaphoreType.DMA,
            pltpu.SemaphoreType.DMA,
            pltpu.SemaphoreType.DMA,
            pltpu.SemaphoreType.DMA,
            pltpu.SemaphoreType.DMA,
            pltpu.SemaphoreType.DMA,
            pltpu.SemaphoreType.DMA,
        ],
        compiler_params=pltpu.CompilerParams(use_tc_tiling_on_sc=False),
    )
    def sc_gather(h_hbm, src_hbm, out_hbm, idxall, rows, *sems):
        sg, sw = sems[:NB], sems[NB:]
        wid = lax.axis_index("s") * NC + lax.axis_index("c")
        base = wid * CPW

        pltpu.sync_copy(src_hbm.at[pl.ds(base, CPW)], idxall)

        def start_g(j, b):
            pltpu.async_copy(h_hbm.at[idxall.at[j]], rows.at[b], sg[b])

        def wait_g(b):
            pltpu.make_async_copy(h_hbm.at[idxall.at[0]], rows.at[b], sg[b]).wait()

        def start_w(j, b):
            pltpu.async_copy(rows.at[b], out_hbm.at[pl.ds((base + j) * CH, CH)],
                             sw[b])

        def wait_w(b):
            pltpu.make_async_copy(rows.at[b], out_hbm.at[pl.ds(0, CH)],
                                  sw[b]).wait()

        for b in range(NB):
            start_g(b, b)

        def body(g, carry):
            for b in range(NB):
                j = g * NB + b
                wait_g(b)
                start_w(j, b)
                wait_w(b)
                start_g(j + NB, b)
            return carry

        lax.fori_loop(0, NGROUPS - 1, body, 0)
        for b in range(NB):
            j = (NGROUPS - 1) * NB + b
            wait_g(b)
            start_w(j, b)
        for b in range(NB):
            wait_w(b)

    @functools.partial(
        pl.kernel, mesh=mesh,
        out_type=jax.ShapeDtypeStruct((NC * N_ACC, H), _f32),
        scratch_types=[
            pltpu.VMEM((CPW, CH), jnp.int32),
            pltpu.VMEM((NB, CH, H), _f32),
            pltpu.VMEM((NSTRIPE, H), _f32),
            pltpu.VMEM_SHARED((N_ACC, H), _f32),
            pltpu.SemaphoreType.DMA,
            pltpu.SemaphoreType.DMA,
            pltpu.SemaphoreType.DMA,
            pltpu.SemaphoreType.DMA,
            pltpu.SemaphoreType.DMA,
            pltpu.SemaphoreType.DMA,
            pltpu.SemaphoreType.DMA,
            pltpu.SemaphoreType.DMA,
        ],
        compiler_params=pltpu.CompilerParams(use_tc_tiling_on_sc=False),
    )
    def sc_scatter(vals_hbm, dst_hbm, zeros_hbm, out_hbm, idxall, vbuf, buf_v,
                   acc, *sems):
        sv, ss = sems[:NB], sems[NB:]
        c = lax.axis_index("c")
        s = lax.axis_index("s")
        wid = s * NC + c
        base = wid * CPW

        pltpu.sync_copy(dst_hbm.at[pl.ds(base, CPW)], idxall)
        pltpu.sync_copy(zeros_hbm.at[pl.ds(s * NSTRIPE, NSTRIPE)], buf_v)
        pltpu.sync_copy(buf_v, acc.at[pl.ds(s * NSTRIPE, NSTRIPE)])
        plsc.subcore_barrier()

        def start_v(j, b):
            pltpu.async_copy(vals_hbm.at[pl.ds((base + j) * CH, CH)],
                             vbuf.at[b], sv[b])

        def wait_v(b):
            pltpu.make_async_copy(vals_hbm.at[pl.ds(0, CH)], vbuf.at[b],
                                  sv[b]).wait()

        def start_s(j, b):
            pltpu.async_copy(vbuf.at[b], acc.at[idxall.at[j]], ss[b], add=True)

        def wait_s(b):
            pltpu.make_async_copy(vbuf.at[b], acc.at[idxall.at[0]],
                                  ss[b]).wait()

        for b in range(NB):
            start_v(b, b)

        def body(g, carry):
            for b in range(NB):
                j = g * NB + b
                wait_v(b)
                start_s(j, b)
                wait_s(b)
                start_v(j + NB, b)
            return carry

        lax.fori_loop(0, NGROUPS - 1, body, 0)
        for b in range(NB):
            j = (NGROUPS - 1) * NB + b
            wait_v(b)
            start_s(j, b)
            wait_s(b)

        plsc.subcore_barrier()
        pltpu.sync_copy(acc.at[pl.ds(s * NSTRIPE, NSTRIPE)], buf_v)
        pltpu.sync_copy(buf_v, out_hbm.at[pl.ds(c * N_ACC + s * NSTRIPE,
                                                NSTRIPE)])

    return sc_gather, sc_scatter


def _sc_gather(h, src2d):
    return _build_sc_kernels()[0](h, src2d)


def _sc_scatter(vals, dst2d, zeros_nh):
    return _build_sc_kernels()[1](vals, dst2d, zeros_nh)


def kernel(x_node, x_edge, edge_index, node_W, node_b, edge_W, edge_b,
           en_W1, en_b1, en_W2, en_b2, gru_Wih, gru_Whh, gru_bih, gru_bhh):
    pad = E_PAD - E
    src = jnp.concatenate([edge_index[0], jnp.zeros((pad,), jnp.int32)])
    dst = jnp.concatenate(
        [edge_index[1], (jnp.arange(pad, dtype=jnp.int32) % N)])
    src = src.reshape(NCHUNK_P, CH)
    dst = dst.reshape(NCHUNK_P, CH)

    node_WT = node_W.T
    node_b2d = node_b.reshape(1, H)
    edge_WT = edge_W.T
    edge_b2d = edge_b.reshape(1, H)
    en_W1T = en_W1.T
    en_b12d = en_b1.reshape(1, H)
    b2c = en_b2.reshape(HH, 1)

    wir = gru_Wih[0:H].T
    wiz = gru_Wih[H:2 * H].T
    win = gru_Wih[2 * H:3 * H].T
    whr = gru_Whh[0:H].T
    whz = gru_Whh[H:2 * H].T
    whn = gru_Whh[2 * H:3 * H].T
    br = (gru_bih[0:H] + gru_bhh[0:H]).reshape(1, H)
    bz = (gru_bih[H:2 * H] + gru_bhh[H:2 * H]).reshape(1, H)
    bin_ = gru_bih[2 * H:3 * H].reshape(1, H)
    bhn = gru_bhh[2 * H:3 * H].reshape(1, H)
    mats = (wir, wiz, win, whr, whz, whn)
    biases = (br, bz, bin_, bhn)

    zeros_nh = jnp.zeros((N_ACC, H), _f32)
    ones_eh = jnp.concatenate(
        [jnp.ones((E, H), _f32), jnp.zeros((pad, H), _f32)])

    h = _embed_node(x_node, node_WT, node_b2d)
    ef = _edge_feat(x_edge, edge_WT, edge_b2d, en_W1T, en_b12d)

    degp = _sc_scatter(ones_eh, dst, zeros_nh)

    for _ in range(N_LAYERS):
        hs = _sc_gather(h, src)
        msg = _msg(hs, ef, en_W2, b2c)
        mp = _sc_scatter(msg, dst, zeros_nh)
        h = _gru(mp, degp, h, mats, biases)
    return h


# speedup vs baseline: 1.1315x; 1.1315x over previous
import functools

import numpy as np
import jax
import jax.numpy as jnp
from jax import lax
from jax.experimental import pallas as pl
from jax.experimental.pallas import tpu as pltpu
from jax.experimental.pallas import tpu_sc as plsc

N = 10000
E = 160000
H = 32
HH = H * H
N_LAYERS = 3

PK = 8
LW = PK * H

NC, NS = 2, 16
NW = NC * NS
CH = 128
CPW = 40
NB = 4
NGROUPS = CPW // NB
NCHUNK_P = NW * CPW
E_PAD = NCHUNK_P * CH
EPK = E_PAD // PK

NPR = N // PK
NPP = 1280
SPR = 128
NPOUT = NS * SPR
ACC_ROWS = NPOUT * PK

BE = 1280
BEP = BE // PK
NBLK_REAL = E // BE
BNP = 256

_f32 = jnp.float32


def _dot(a, b):
    return jnp.dot(a, b, preferred_element_type=_f32)


def _embed_node_body(x_ref, w_ref, b_ref, o_ref):
    o_ref[...] = _dot(x_ref[...], w_ref[...]) + b_ref[...]


def _embed_node(x8, w8, b8):
    grid = NPP // BNP
    return pl.pallas_call(
        _embed_node_body,
        grid=(grid,),
        in_specs=[
            pl.BlockSpec((BNP, x8.shape[1]), lambda i: (i, 0)),
            pl.BlockSpec(w8.shape, lambda i: (0, 0)),
            pl.BlockSpec(b8.shape, lambda i: (0, 0)),
        ],
        out_specs=pl.BlockSpec((BNP, LW), lambda i: (i, 0)),
        out_shape=jax.ShapeDtypeStruct((NPP, LW), _f32),
    )(x8, w8, b8)


def _edge_feat_body(x_ref, w_ref, b_ref, w1_ref, b1_ref, o_ref):
    t = _dot(x_ref[...], w_ref[...]) + b_ref[...]
    o_ref[...] = jnp.maximum(_dot(t, w1_ref[...]) + b1_ref[...], 0.0)


def _edge_feat(x8, w8, b8, w18, b18):
    grid = E_PAD // BE
    return pl.pallas_call(
        _edge_feat_body,
        grid=(grid,),
        in_specs=[
            pl.BlockSpec((BEP, x8.shape[1]),
                         lambda i: (jnp.minimum(i, NBLK_REAL - 1), 0)),
            pl.BlockSpec(w8.shape, lambda i: (0, 0)),
            pl.BlockSpec(b8.shape, lambda i: (0, 0)),
            pl.BlockSpec(w18.shape, lambda i: (0, 0)),
            pl.BlockSpec(b18.shape, lambda i: (0, 0)),
        ],
        out_specs=pl.BlockSpec((BEP, LW), lambda i: (i, 0)),
        out_shape=jax.ShapeDtypeStruct((EPK, LW), _f32),
    )(x8, w8, b8, w18, b18)


def _msg_body(hs_ref, ef_ref, w2_ref, b2c_ref, o_ref):
    @pl.when(pl.program_id(0) >= NBLK_REAL)
    def _():
        o_ref[...] = jnp.zeros_like(o_ref)

    @pl.when(pl.program_id(0) < NBLK_REAL)
    def _():
        hsT = hs_ref[...].T
        efT = ef_ref[...].T
        parts = []
        for q in range(PK):
            efq = efT[q * H:(q + 1) * H, :]
            gq = lax.dot_general(w2_ref[...], efq, (((1,), (0,)), ((), ())),
                                 preferred_element_type=_f32) + b2c_ref[...]
            hq = hsT[q * H:(q + 1) * H, :]
            acc = gq[0:H, :] * hq[0:1, :]
            for i in range(1, H):
                acc += gq[i * H:(i + 1) * H, :] * hq[i:i + 1, :]
            parts.append(acc)
        o_ref[...] = jnp.concatenate(parts, axis=0).T


def _msg(hs, ef, w2, b2c):
    grid = E_PAD // BE
    blk = pl.BlockSpec((BEP, LW), lambda i: (i, 0))
    return pl.pallas_call(
        _msg_body,
        grid=(grid,),
        in_specs=[
            blk,
            blk,
            pl.BlockSpec(w2.shape, lambda i: (0, 0)),
            pl.BlockSpec(b2c.shape, lambda i: (0, 0)),
        ],
        out_specs=blk,
        out_shape=jax.ShapeDtypeStruct((EPK, LW), _f32),
    )(hs, ef, w2, b2c)


def _gru_body(p0_ref, p1_ref, d0_ref, d1_ref, h_ref,
              wir_ref, wiz_ref, win_ref, whr_ref, whz_ref, whn_ref,
              br_ref, bz_ref, bin_ref, bhn_ref, o_ref):
    deg = jnp.maximum(d0_ref[0] + d1_ref[0], 1.0)
    m = jnp.maximum((p0_ref[0] + p1_ref[0]) / deg, 0.0)
    h = h_ref[...]
    r = jax.nn.sigmoid(_dot(m, wir_ref[...]) + _dot(h, whr_ref[...]) + br_ref[...])
    z = jax.nn.sigmoid(_dot(m, wiz_ref[...]) + _dot(h, whz_ref[...]) + bz_ref[...])
    n = jnp.tanh(_dot(m, win_ref[...]) + bin_ref[...]
                 + r * (_dot(h, whn_ref[...]) + bhn_ref[...]))
    o_ref[...] = (1.0 - z) * n + z * h


def _gru(mp, degp, h, mats, biases):
    grid = NPP // BNP
    blk = pl.BlockSpec((BNP, LW), lambda i: (i, 0))
    p0 = pl.BlockSpec((1, BNP, LW), lambda i: (0, i, 0))
    p1 = pl.BlockSpec((1, BNP, LW), lambda i: (1, i, 0))
    full = lambda a: pl.BlockSpec(a.shape, lambda i: (0, 0))
    return pl.pallas_call(
        _gru_body,
        grid=(grid,),
        in_specs=[p0, p1, p0, p1, blk]
        + [full(m) for m in mats] + [full(b) for b in biases],
        out_specs=blk,
        out_shape=jax.ShapeDtypeStruct((NPP, LW), _f32),
    )(mp, mp, degp, degp, h, *mats, *biases)


@functools.cache
def _build_sc_kernels():
    mesh = plsc.VectorSubcoreMesh(core_axis_name="c", subcore_axis_name="s",
                                  num_cores=NC, num_subcores=NS)
    dma_sems = [pltpu.SemaphoreType.DMA] * (2 * NB)

    @functools.partial(
        pl.kernel, mesh=mesh,
        out_type=jax.ShapeDtypeStruct((NCHUNK_P, CH, H), _f32),
        scratch_types=[
            pltpu.VMEM((CPW, CH), jnp.int32),
            pltpu.VMEM((NB, CH, H), _f32),
        ] + dma_sems,
        compiler_params=pltpu.CompilerParams(use_tc_tiling_on_sc=False),
    )
    def sc_gather(h_hbm, src_hbm, out_hbm, idxall, rows, *sems):
        sg, sw = sems[:NB], sems[NB:]
        wid = lax.axis_index("s") * NC + lax.axis_index("c")
        base = wid * CPW

        pltpu.sync_copy(src_hbm.at[pl.ds(base, CPW)], idxall)

        def start_g(j, b):
            pltpu.async_copy(h_hbm.at[idxall.at[j]], rows.at[b], sg[b])

        def wait_g(b):
            pltpu.make_async_copy(h_hbm.at[idxall.at[0]], rows.at[b],
                                  sg[b]).wait()

        def start_w(j, b):
            pltpu.async_copy(rows.at[b], out_hbm.at[base + j], sw[b])

        def wait_w(b):
            pltpu.make_async_copy(rows.at[b], out_hbm.at[0], sw[b]).wait()

        for b in range(NB):
            start_g(b, b)

        def body(g, carry):
            for b in range(NB):
                j = g * NB + b
                wait_g(b)
                start_w(j, b)
                wait_w(b)
                start_g(j + NB, b)
            return carry

        lax.fori_loop(0, NGROUPS - 1, body, 0)
        for b in range(NB):
            j = (NGROUPS - 1) * NB + b
            wait_g(b)
            start_w(j, b)
        for b in range(NB):
            wait_w(b)

    @functools.partial(
        pl.kernel, mesh=mesh,
        out_type=jax.ShapeDtypeStruct((NC, NPOUT * PK, H), _f32),
        scratch_types=[
            pltpu.VMEM((CPW, CH), jnp.int32),
            pltpu.VMEM((NB, CH, H), _f32),
            pltpu.VMEM((SPR * PK, H), _f32),
            pltpu.VMEM_SHARED((ACC_ROWS, H), _f32),
        ] + dma_sems,
        compiler_params=pltpu.CompilerParams(use_tc_tiling_on_sc=False),
    )
    def sc_scatter(vals_hbm, dst_hbm, out_hbm, idxall, vbuf, buf_v, acc,
                   *sems):
        sv, ss = sems[:NB], sems[NB:]
        c = lax.axis_index("c")
        s = lax.axis_index("s")
        wid = s * NC + c
        base = wid * CPW

        pltpu.sync_copy(dst_hbm.at[pl.ds(base, CPW)], idxall)

        z = jnp.zeros((16,), _f32)

        def zbody(r, carry):
            buf_v[r, pl.ds(0, 16)] = z
            buf_v[r, pl.ds(16, 16)] = z
            return carry

        lax.fori_loop(0, SPR * PK, zbody, 0)
        pltpu.sync_copy(buf_v, acc.at[pl.ds(s * SPR * PK, SPR * PK)])
        plsc.subcore_barrier()

        def start_v(j, b):
            pltpu.async_copy(vals_hbm.at[base + j], vbuf.at[b], sv[b])

        def wait_v(b):
            pltpu.make_async_copy(vals_hbm.at[0], vbuf.at[b], sv[b]).wait()

        def start_s(j, b):
            pltpu.async_copy(vbuf.at[b], acc.at[idxall.at[j]], ss[b],
                             add=True)

        def wait_s(b):
            pltpu.make_async_copy(vbuf.at[b], acc.at[idxall.at[0]],
                                  ss[b]).wait()

        for b in range(NB):
            start_v(b, b)

        def body(g, carry):
            for b in range(NB):
                j = g * NB + b
                wait_v(b)
                start_s(j, b)
                wait_s(b)
                start_v(j + NB, b)
            return carry

        lax.fori_loop(0, NGROUPS - 1, body, 0)
        for b in range(NB):
            j = (NGROUPS - 1) * NB + b
            wait_v(b)
            start_s(j, b)
            wait_s(b)

        plsc.subcore_barrier()
        pltpu.sync_copy(acc.at[pl.ds(s * SPR * PK, SPR * PK)], buf_v)
        pltpu.sync_copy(buf_v, out_hbm.at[c].at[pl.ds(s * SPR * PK, SPR * PK)])

    return sc_gather, sc_scatter


def _sc_gather(h32, src2d):
    return _build_sc_kernels()[0](h32, src2d).reshape(EPK, LW)


def _sc_scatter(vals_p, dst2d):
    out = _build_sc_kernels()[1](vals_p.reshape(NCHUNK_P, CH, H), dst2d)
    return out.reshape(NC, NPOUT, LW)


def _kron8(w):
    return jnp.kron(jnp.eye(PK, dtype=_f32), w)


def kernel(x_node, x_edge, edge_index, node_W, node_b, edge_W, edge_b,
           en_W1, en_b1, en_W2, en_b2, gru_Wih, gru_Whh, gru_bih, gru_bhh):
    pad = E_PAD - E
    src = jnp.concatenate([edge_index[0], jnp.zeros((pad,), jnp.int32)])
    dst = jnp.concatenate(
        [edge_index[1], (jnp.arange(pad, dtype=jnp.int32) % N)])
    src = src.reshape(NCHUNK_P, CH)
    dst = dst.reshape(NCHUNK_P, CH)

    nfe = x_edge.shape[1]
    x8e = jnp.concatenate(
        [x_edge.reshape(E // PK, PK * nfe),
         jnp.zeros((EPK - E // PK, PK * nfe), _f32)])
    nfn = x_node.shape[1]
    x8n = jnp.concatenate(
        [x_node.reshape(NPR, PK * nfn),
         jnp.zeros((NPP - NPR, PK * nfn), _f32)])

    t8 = lambda b: jnp.tile(b.reshape(1, H), (1, PK))
    node_W8 = _kron8(node_W.T)
    node_b8 = t8(node_b)
    edge_W8 = _kron8(edge_W.T)
    edge_b8 = t8(edge_b)
    en_W18 = _kron8(en_W1.T)
    en_b18 = t8(en_b1)
    b2c = en_b2.reshape(HH, 1)

    mats = tuple(_kron8(m) for m in (
        gru_Wih[0:H].T, gru_Wih[H:2 * H].T, gru_Wih[2 * H:3 * H].T,
        gru_Whh[0:H].T, gru_Whh[H:2 * H].T, gru_Whh[2 * H:3 * H].T))
    biases = (t8(gru_bih[0:H] + gru_bhh[0:H]),
              t8(gru_bih[H:2 * H] + gru_bhh[H:2 * H]),
              t8(gru_bih[2 * H:3 * H]),
              t8(gru_bhh[2 * H:3 * H]))

    ones_p = jnp.concatenate(
        [jnp.ones((E // PK, LW), _f32), jnp.zeros((EPK - E // PK, LW), _f32)])

    h = _embed_node(x8n, node_W8, node_b8)
    ef = _edge_feat(x8e, edge_W8, edge_b8, en_W18, en_b18)

    degp = _sc_scatter(ones_p, dst)

    for _ in range(N_LAYERS):
        hs = _sc_gather(h.reshape(NPP * PK, H), src)
        msg = _msg(hs, ef, en_W2, b2c)
        mp = _sc_scatter(msg, dst)
        h = _gru(mp, degp, h, mats, biases)
    return h.reshape(NPP * PK, H)[:N]


# speedup vs baseline: 1.1323x; 1.0007x over previous
import functools

import numpy as np
import jax
import jax.numpy as jnp
from jax import lax
from jax.experimental import pallas as pl
from jax.experimental.pallas import tpu as pltpu
from jax.experimental.pallas import tpu_sc as plsc

N = 10000
E = 160000
H = 32
HH = H * H
N_LAYERS = 3

PK = 8
LW = PK * H

NC, NS = 2, 16
NW = NC * NS
CH = 128
CPW = 40
NB = 8
NGROUPS = CPW // NB
NCHUNK_P = NW * CPW
E_PAD = NCHUNK_P * CH
EPK = E_PAD // PK

NPR = N // PK
NPP = 1280
SPR = 128
NPOUT = NS * SPR
ACC_ROWS = NPOUT * PK

BE = 1280
BEP = BE // PK
NBLK_REAL = E // BE
BNP = 256

_f32 = jnp.float32


def _dot(a, b):
    return jnp.dot(a, b, preferred_element_type=_f32)


def _embed_node_body(x_ref, w_ref, b_ref, o_ref):
    o_ref[...] = _dot(x_ref[...], w_ref[...]) + b_ref[...]


def _embed_node(x8, w8, b8):
    grid = NPP // BNP
    return pl.pallas_call(
        _embed_node_body,
        grid=(grid,),
        in_specs=[
            pl.BlockSpec((BNP, x8.shape[1]), lambda i: (i, 0)),
            pl.BlockSpec(w8.shape, lambda i: (0, 0)),
            pl.BlockSpec(b8.shape, lambda i: (0, 0)),
        ],
        out_specs=pl.BlockSpec((BNP, LW), lambda i: (i, 0)),
        out_shape=jax.ShapeDtypeStruct((NPP, LW), _f32),
    )(x8, w8, b8)


def _edge_feat_body(x_ref, w_ref, b_ref, w1_ref, b1_ref, o_ref):
    t = _dot(x_ref[...], w_ref[...]) + b_ref[...]
    o_ref[...] = jnp.maximum(_dot(t, w1_ref[...]) + b1_ref[...], 0.0)


def _edge_feat(x8, w8, b8, w18, b18):
    grid = E_PAD // BE
    return pl.pallas_call(
        _edge_feat_body,
        grid=(grid,),
        in_specs=[
            pl.BlockSpec((BEP, x8.shape[1]),
                         lambda i: (jnp.minimum(i, NBLK_REAL - 1), 0)),
            pl.BlockSpec(w8.shape, lambda i: (0, 0)),
            pl.BlockSpec(b8.shape, lambda i: (0, 0)),
            pl.BlockSpec(w18.shape, lambda i: (0, 0)),
            pl.BlockSpec(b18.shape, lambda i: (0, 0)),
        ],
        out_specs=pl.BlockSpec((BEP, LW), lambda i: (i, 0)),
        out_shape=jax.ShapeDtypeStruct((EPK, LW), _f32),
    )(x8, w8, b8, w18, b18)


def _msg_body(hs_ref, ef_ref, w2_ref, b2c_ref, o_ref):
    @pl.when(pl.program_id(0) >= NBLK_REAL)
    def _():
        o_ref[...] = jnp.zeros_like(o_ref)

    @pl.when(pl.program_id(0) < NBLK_REAL)
    def _():
        hsT = hs_ref[...].T
        efT = ef_ref[...].T.astype(jnp.bfloat16)
        parts = []
        for q in range(PK):
            efq = efT[q * H:(q + 1) * H, :]
            gq = lax.dot_general(w2_ref[...], efq, (((1,), (0,)), ((), ())),
                                 preferred_element_type=_f32) + b2c_ref[...]
            hq = hsT[q * H:(q + 1) * H, :]
            acc = gq[0:H, :] * hq[0:1, :]
            for i in range(1, H):
                acc += gq[i * H:(i + 1) * H, :] * hq[i:i + 1, :]
            parts.append(acc)
        o_ref[...] = jnp.concatenate(parts, axis=0).T


def _msg(hs, ef, w2, b2c):
    grid = E_PAD // BE
    blk = pl.BlockSpec((BEP, LW), lambda i: (i, 0))
    return pl.pallas_call(
        _msg_body,
        grid=(grid,),
        in_specs=[
            blk,
            blk,
            pl.BlockSpec(w2.shape, lambda i: (0, 0)),
            pl.BlockSpec(b2c.shape, lambda i: (0, 0)),
        ],
        out_specs=blk,
        out_shape=jax.ShapeDtypeStruct((EPK, LW), _f32),
    )(hs, ef, w2, b2c)


def _gru_body(p0_ref, p1_ref, d0_ref, d1_ref, h_ref,
              wir_ref, wiz_ref, win_ref, whr_ref, whz_ref, whn_ref,
              br_ref, bz_ref, bin_ref, bhn_ref, o_ref):
    deg = jnp.maximum(d0_ref[0] + d1_ref[0], 1.0)
    m = jnp.maximum((p0_ref[0] + p1_ref[0]) / deg, 0.0)
    h = h_ref[...]
    r = jax.nn.sigmoid(_dot(m, wir_ref[...]) + _dot(h, whr_ref[...]) + br_ref[...])
    z = jax.nn.sigmoid(_dot(m, wiz_ref[...]) + _dot(h, whz_ref[...]) + bz_ref[...])
    n = jnp.tanh(_dot(m, win_ref[...]) + bin_ref[...]
                 + r * (_dot(h, whn_ref[...]) + bhn_ref[...]))
    o_ref[...] = (1.0 - z) * n + z * h


def _gru(mp, degp, h, mats, biases):
    grid = NPP // BNP
    blk = pl.BlockSpec((BNP, LW), lambda i: (i, 0))
    p0 = pl.BlockSpec((1, BNP, LW), lambda i: (0, i, 0))
    p1 = pl.BlockSpec((1, BNP, LW), lambda i: (1, i, 0))
    full = lambda a: pl.BlockSpec(a.shape, lambda i: (0, 0))
    return pl.pallas_call(
        _gru_body,
        grid=(grid,),
        in_specs=[p0, p1, p0, p1, blk]
        + [full(m) for m in mats] + [full(b) for b in biases],
        out_specs=blk,
        out_shape=jax.ShapeDtypeStruct((NPP, LW), _f32),
    )(mp, mp, degp, degp, h, *mats, *biases)


@functools.cache
def _build_sc_kernels():
    mesh = plsc.VectorSubcoreMesh(core_axis_name="c", subcore_axis_name="s",
                                  num_cores=NC, num_subcores=NS)
    dma_sems = [pltpu.SemaphoreType.DMA] * (2 * NB)

    @functools.partial(
        pl.kernel, mesh=mesh,
        out_type=jax.ShapeDtypeStruct((NCHUNK_P, CH, H), _f32),
        scratch_types=[
            pltpu.VMEM((CPW, CH), jnp.int32),
            pltpu.VMEM((NB, CH, H), _f32),
        ] + dma_sems,
        compiler_params=pltpu.CompilerParams(use_tc_tiling_on_sc=False),
    )
    def sc_gather(h_hbm, src_hbm, out_hbm, idxall, rows, *sems):
        sg, sw = sems[:NB], sems[NB:]
        wid = lax.axis_index("s") * NC + lax.axis_index("c")
        base = wid * CPW

        pltpu.sync_copy(src_hbm.at[pl.ds(base, CPW)], idxall)

        def start_g(j, b):
            pltpu.async_copy(h_hbm.at[idxall.at[j]], rows.at[b], sg[b])

        def wait_g(b):
            pltpu.make_async_copy(h_hbm.at[idxall.at[0]], rows.at[b],
                                  sg[b]).wait()

        def start_w(j, b):
            pltpu.async_copy(rows.at[b], out_hbm.at[base + j], sw[b])

        def wait_w(b):
            pltpu.make_async_copy(rows.at[b], out_hbm.at[0], sw[b]).wait()

        for b in range(NB):
            start_g(b, b)

        def body(g, carry):
            for b in range(NB):
                j = g * NB + b
                wait_g(b)
                start_w(j, b)
                wait_w(b)
                start_g(j + NB, b)
            return carry

        lax.fori_loop(0, NGROUPS - 1, body, 0)
        for b in range(NB):
            j = (NGROUPS - 1) * NB + b
            wait_g(b)
            start_w(j, b)
        for b in range(NB):
            wait_w(b)

    @functools.partial(
        pl.kernel, mesh=mesh,
        out_type=jax.ShapeDtypeStruct((NC, NPOUT * PK, H), _f32),
        scratch_types=[
            pltpu.VMEM((CPW, CH), jnp.int32),
            pltpu.VMEM((NB, CH, H), _f32),
            pltpu.VMEM((SPR * PK, H), _f32),
            pltpu.VMEM_SHARED((ACC_ROWS, H), _f32),
        ] + dma_sems,
        compiler_params=pltpu.CompilerParams(use_tc_tiling_on_sc=False),
    )
    def sc_scatter(vals_hbm, dst_hbm, out_hbm, idxall, vbuf, buf_v, acc,
                   *sems):
        sv, ss = sems[:NB], sems[NB:]
        c = lax.axis_index("c")
        s = lax.axis_index("s")
        wid = s * NC + c
        base = wid * CPW

        pltpu.sync_copy(dst_hbm.at[pl.ds(base, CPW)], idxall)

        z = jnp.zeros((16,), _f32)

        def zbody(r, carry):
            buf_v[r, pl.ds(0, 16)] = z
            buf_v[r, pl.ds(16, 16)] = z
            return carry

        lax.fori_loop(0, SPR * PK, zbody, 0)
        pltpu.sync_copy(buf_v, acc.at[pl.ds(s * SPR * PK, SPR * PK)])
        plsc.subcore_barrier()

        def start_v(j, b):
            pltpu.async_copy(vals_hbm.at[base + j], vbuf.at[b], sv[b])

        def wait_v(b):
            pltpu.make_async_copy(vals_hbm.at[0], vbuf.at[b], sv[b]).wait()

        def start_s(j, b):
            pltpu.async_copy(vbuf.at[b], acc.at[idxall.at[j]], ss[b],
                             add=True)

        def wait_s(b):
            pltpu.make_async_copy(vbuf.at[b], acc.at[idxall.at[0]],
                                  ss[b]).wait()

        for b in range(NB):
            start_v(b, b)

        def body(g, carry):
            for b in range(NB):
                j = g * NB + b
                wait_v(b)
                start_s(j, b)
                wait_s(b)
                start_v(j + NB, b)
            return carry

        lax.fori_loop(0, NGROUPS - 1, body, 0)
        for b in range(NB):
            j = (NGROUPS - 1) * NB + b
            wait_v(b)
            start_s(j, b)
            wait_s(b)

        plsc.subcore_barrier()
        pltpu.sync_copy(acc.at[pl.ds(s * SPR * PK, SPR * PK)], buf_v)
        pltpu.sync_copy(buf_v, out_hbm.at[c].at[pl.ds(s * SPR * PK, SPR * PK)])

    return sc_gather, sc_scatter


def _sc_gather(h32, src2d):
    return _build_sc_kernels()[0](h32, src2d).reshape(EPK, LW)


def _sc_scatter(vals_p, dst2d):
    out = _build_sc_kernels()[1](vals_p.reshape(NCHUNK_P, CH, H), dst2d)
    return out.reshape(NC, NPOUT, LW)


def _kron8(w):
    return jnp.kron(jnp.eye(PK, dtype=_f32), w)


def kernel(x_node, x_edge, edge_index, node_W, node_b, edge_W, edge_b,
           en_W1, en_b1, en_W2, en_b2, gru_Wih, gru_Whh, gru_bih, gru_bhh):
    pad = E_PAD - E
    src = jnp.concatenate([edge_index[0], jnp.zeros((pad,), jnp.int32)])
    dst = jnp.concatenate(
        [edge_index[1], (jnp.arange(pad, dtype=jnp.int32) % N)])
    src = src.reshape(NCHUNK_P, CH)
    dst = dst.reshape(NCHUNK_P, CH)

    nfe = x_edge.shape[1]
    x8e = jnp.concatenate(
        [x_edge.reshape(E // PK, PK * nfe),
         jnp.zeros((EPK - E // PK, PK * nfe), _f32)])
    nfn = x_node.shape[1]
    x8n = jnp.concatenate(
        [x_node.reshape(NPR, PK * nfn),
         jnp.zeros((NPP - NPR, PK * nfn), _f32)])

    t8 = lambda b: jnp.tile(b.reshape(1, H), (1, PK))
    node_W8 = _kron8(node_W.T)
    node_b8 = t8(node_b)
    edge_W8 = _kron8(edge_W.T)
    edge_b8 = t8(edge_b)
    en_W18 = _kron8(en_W1.T)
    en_b18 = t8(en_b1)
    b2c = en_b2.reshape(HH, 1)
    en_W2b = en_W2.astype(jnp.bfloat16)

    mats = tuple(_kron8(m) for m in (
        gru_Wih[0:H].T, gru_Wih[H:2 * H].T, gru_Wih[2 * H:3 * H].T,
        gru_Whh[0:H].T, gru_Whh[H:2 * H].T, gru_Whh[2 * H:3 * H].T))
    biases = (t8(gru_bih[0:H] + gru_bhh[0:H]),
              t8(gru_bih[H:2 * H] + gru_bhh[H:2 * H]),
              t8(gru_bih[2 * H:3 * H]),
              t8(gru_bhh[2 * H:3 * H]))

    ones_p = jnp.concatenate(
        [jnp.ones((E // PK, LW), _f32), jnp.zeros((EPK - E // PK, LW), _f32)])

    h = _embed_node(x8n, node_W8, node_b8)
    ef = _edge_feat(x8e, edge_W8, edge_b8, en_W18, en_b18)

    degp = _sc_scatter(ones_p, dst)

    for _ in range(N_LAYERS):
        hs = _sc_gather(h.reshape(NPP * PK, H), src)
        msg = _msg(hs, ef, en_W2b, b2c)
        mp = _sc_scatter(msg, dst)
        h = _gru(mp, degp, h, mats, biases)
    return h.reshape(NPP * PK, H)[:N]


# speedup vs baseline: 1.1575x; 1.0222x over previous
import functools

import numpy as np
import jax
import jax.numpy as jnp
from jax import lax
from jax.experimental import pallas as pl
from jax.experimental.pallas import tpu as pltpu
from jax.experimental.pallas import tpu_sc as plsc

N = 10000
E = 160000
H = 32
HH = H * H
N_LAYERS = 3

PK = 8
LW = PK * H

NC, NS = 2, 16
NW = NC * NS
CH = 128
CPW = 40
NB = 8
NGROUPS = CPW // NB
NCHUNK_P = NW * CPW
E_PAD = NCHUNK_P * CH
EPK = E_PAD // PK

NPR = N // PK
NPP = 1280
SPR = 128
NPOUT = NS * SPR
ACC_ROWS = NPOUT * PK

BE = 1280
BEP = BE // PK
NBLK_REAL = E // BE
BNP = 256

_f32 = jnp.float32


def _dot(a, b):
    return jnp.dot(a, b, preferred_element_type=_f32)


def _embed_node_body(x_ref, w_ref, b_ref, o_ref):
    o_ref[...] = _dot(x_ref[...], w_ref[...]) + b_ref[...]


def _embed_node(x8, w8, b8):
    grid = NPP // BNP
    return pl.pallas_call(
        _embed_node_body,
        grid=(grid,),
        in_specs=[
            pl.BlockSpec((BNP, x8.shape[1]), lambda i: (i, 0)),
            pl.BlockSpec(w8.shape, lambda i: (0, 0)),
            pl.BlockSpec(b8.shape, lambda i: (0, 0)),
        ],
        out_specs=pl.BlockSpec((BNP, LW), lambda i: (i, 0)),
        out_shape=jax.ShapeDtypeStruct((NPP, LW), _f32),
    )(x8, w8, b8)


def _edge_feat_body(x_ref, w_ref, b_ref, w1_ref, b1_ref, o_ref):
    t = _dot(x_ref[...], w_ref[...]) + b_ref[...]
    o_ref[...] = jnp.maximum(_dot(t, w1_ref[...]) + b1_ref[...], 0.0)


def _edge_feat(x8, w8, b8, w18, b18):
    grid = E_PAD // BE
    return pl.pallas_call(
        _edge_feat_body,
        grid=(grid,),
        in_specs=[
            pl.BlockSpec((BEP, x8.shape[1]),
                         lambda i: (jnp.minimum(i, NBLK_REAL - 1), 0)),
            pl.BlockSpec(w8.shape, lambda i: (0, 0)),
            pl.BlockSpec(b8.shape, lambda i: (0, 0)),
            pl.BlockSpec(w18.shape, lambda i: (0, 0)),
            pl.BlockSpec(b18.shape, lambda i: (0, 0)),
        ],
        out_specs=pl.BlockSpec((BEP, LW), lambda i: (i, 0)),
        out_shape=jax.ShapeDtypeStruct((EPK, LW), _f32),
    )(x8, w8, b8, w18, b18)


def _msg_body(hs_ref, ef_ref, w2_ref, b28_ref, o_ref):
    @pl.when(pl.program_id(0) >= NBLK_REAL)
    def _():
        o_ref[...] = jnp.zeros_like(o_ref)

    @pl.when(pl.program_id(0) < NBLK_REAL)
    def _():
        hs = hs_ref[...]
        hsT = hs.T
        efT = ef_ref[...].T.astype(jnp.bfloat16)
        parts = []
        for q in range(PK):
            efq = efT[q * H:(q + 1) * H, :]
            gq = lax.dot_general(w2_ref[...], efq, (((1,), (0,)), ((), ())),
                                 preferred_element_type=_f32)
            hq = hsT[q * H:(q + 1) * H, :]
            acc = gq[0:H, :] * hq[0:1, :]
            for i in range(1, H):
                acc += gq[i * H:(i + 1) * H, :] * hq[i:i + 1, :]
            parts.append(acc)
        o_ref[...] = jnp.concatenate(parts, axis=0).T + _dot(hs, b28_ref[...])


def _msg(hs, ef, w2, b28):
    grid = E_PAD // BE
    blk = pl.BlockSpec((BEP, LW), lambda i: (i, 0))
    return pl.pallas_call(
        _msg_body,
        grid=(grid,),
        in_specs=[
            blk,
            blk,
            pl.BlockSpec(w2.shape, lambda i: (0, 0)),
            pl.BlockSpec(b28.shape, lambda i: (0, 0)),
        ],
        out_specs=blk,
        out_shape=jax.ShapeDtypeStruct((EPK, LW), _f32),
    )(hs, ef, w2, b28)


def _gru_body(p0_ref, p1_ref, d0_ref, d1_ref, h_ref,
              wir_ref, wiz_ref, win_ref, whr_ref, whz_ref, whn_ref,
              br_ref, bz_ref, bin_ref, bhn_ref, o_ref):
    deg = jnp.maximum(d0_ref[0] + d1_ref[0], 1.0)
    m = jnp.maximum((p0_ref[0] + p1_ref[0]) / deg, 0.0)
    h = h_ref[...]
    r = jax.nn.sigmoid(_dot(m, wir_ref[...]) + _dot(h, whr_ref[...]) + br_ref[...])
    z = jax.nn.sigmoid(_dot(m, wiz_ref[...]) + _dot(h, whz_ref[...]) + bz_ref[...])
    n = jnp.tanh(_dot(m, win_ref[...]) + bin_ref[...]
                 + r * (_dot(h, whn_ref[...]) + bhn_ref[...]))
    o_ref[...] = (1.0 - z) * n + z * h


def _gru(mp, degp, h, mats, biases):
    grid = NPP // BNP
    blk = pl.BlockSpec((BNP, LW), lambda i: (i, 0))
    p0 = pl.BlockSpec((1, BNP, LW), lambda i: (0, i, 0))
    p1 = pl.BlockSpec((1, BNP, LW), lambda i: (1, i, 0))
    full = lambda a: pl.BlockSpec(a.shape, lambda i: (0, 0))
    return pl.pallas_call(
        _gru_body,
        grid=(grid,),
        in_specs=[p0, p1, p0, p1, blk]
        + [full(m) for m in mats] + [full(b) for b in biases],
        out_specs=blk,
        out_shape=jax.ShapeDtypeStruct((NPP, LW), _f32),
    )(mp, mp, degp, degp, h, *mats, *biases)


@functools.cache
def _build_sc_kernels():
    mesh = plsc.VectorSubcoreMesh(core_axis_name="c", subcore_axis_name="s",
                                  num_cores=NC, num_subcores=NS)
    dma_sems = [pltpu.SemaphoreType.DMA] * (2 * NB)

    @functools.partial(
        pl.kernel, mesh=mesh,
        out_type=jax.ShapeDtypeStruct((NCHUNK_P, CH, H), _f32),
        scratch_types=[
            pltpu.VMEM((CPW, CH), jnp.int32),
            pltpu.VMEM((NB, CH, H), _f32),
        ] + dma_sems,
        compiler_params=pltpu.CompilerParams(use_tc_tiling_on_sc=False),
    )
    def sc_gather(h_hbm, src_hbm, out_hbm, idxall, rows, *sems):
        sg, sw = sems[:NB], sems[NB:]
        wid = lax.axis_index("s") * NC + lax.axis_index("c")
        base = wid * CPW

        pltpu.sync_copy(src_hbm.at[pl.ds(base, CPW)], idxall)

        def start_g(j, b):
            pltpu.async_copy(h_hbm.at[idxall.at[j]], rows.at[b], sg[b])

        def wait_g(b):
            pltpu.make_async_copy(h_hbm.at[idxall.at[0]], rows.at[b],
                                  sg[b]).wait()

        def start_w(j, b):
            pltpu.async_copy(rows.at[b], out_hbm.at[base + j], sw[b])

        def wait_w(b):
            pltpu.make_async_copy(rows.at[b], out_hbm.at[0], sw[b]).wait()

        for b in range(NB):
            start_g(b, b)

        def body(g, carry):
            for b in range(NB):
                j = g * NB + b
                wait_g(b)
                start_w(j, b)
                wait_w(b)
                start_g(j + NB, b)
            return carry

        lax.fori_loop(0, NGROUPS - 1, body, 0)
        for b in range(NB):
            j = (NGROUPS - 1) * NB + b
            wait_g(b)
            start_w(j, b)
        for b in range(NB):
            wait_w(b)

    @functools.partial(
        pl.kernel, mesh=mesh,
        out_type=jax.ShapeDtypeStruct((NC, NPOUT * PK, H), _f32),
        scratch_types=[
            pltpu.VMEM((CPW, CH), jnp.int32),
            pltpu.VMEM((NB, CH, H), _f32),
            pltpu.VMEM((SPR * PK, H), _f32),
            pltpu.VMEM_SHARED((ACC_ROWS, H), _f32),
        ] + dma_sems,
        compiler_params=pltpu.CompilerParams(use_tc_tiling_on_sc=False),
    )
    def sc_scatter(vals_hbm, dst_hbm, out_hbm, idxall, vbuf, buf_v, acc,
                   *sems):
        sv, ss = sems[:NB], sems[NB:]
        c = lax.axis_index("c")
        s = lax.axis_index("s")
        wid = s * NC + c
        base = wid * CPW

        pltpu.sync_copy(dst_hbm.at[pl.ds(base, CPW)], idxall)

        z = jnp.zeros((16,), _f32)

        def zbody(r, carry):
            buf_v[r, pl.ds(0, 16)] = z
            buf_v[r, pl.ds(16, 16)] = z
            return carry

        lax.fori_loop(0, SPR * PK, zbody, 0)
        pltpu.sync_copy(buf_v, acc.at[pl.ds(s * SPR * PK, SPR * PK)])
        plsc.subcore_barrier()

        def start_v(j, b):
            pltpu.async_copy(vals_hbm.at[base + j], vbuf.at[b], sv[b])

        def wait_v(b):
            pltpu.make_async_copy(vals_hbm.at[0], vbuf.at[b], sv[b]).wait()

        def start_s(j, b):
            pltpu.async_copy(vbuf.at[b], acc.at[idxall.at[j]], ss[b],
                             add=True)

        def wait_s(b):
            pltpu.make_async_copy(vbuf.at[b], acc.at[idxall.at[0]],
                                  ss[b]).wait()

        for b in range(NB):
            start_v(b, b)

        def body(g, carry):
            for b in range(NB):
                j = g * NB + b
                wait_v(b)
                start_s(j, b)
                wait_s(b)
                start_v(j + NB, b)
            return carry

        lax.fori_loop(0, NGROUPS - 1, body, 0)
        for b in range(NB):
            j = (NGROUPS - 1) * NB + b
            wait_v(b)
            start_s(j, b)
            wait_s(b)

        plsc.subcore_barrier()
        pltpu.sync_copy(acc.at[pl.ds(s * SPR * PK, SPR * PK)], buf_v)
        pltpu.sync_copy(buf_v, out_hbm.at[c].at[pl.ds(s * SPR * PK, SPR * PK)])

    return sc_gather, sc_scatter


def _sc_gather(h32, src2d):
    return _build_sc_kernels()[0](h32, src2d).reshape(EPK, LW)


def _sc_scatter(vals_p, dst2d):
    out = _build_sc_kernels()[1](vals_p.reshape(NCHUNK_P, CH, H), dst2d)
    return out.reshape(NC, NPOUT, LW)


def _kron8(w):
    return jnp.kron(jnp.eye(PK, dtype=_f32), w)


def kernel(x_node, x_edge, edge_index, node_W, node_b, edge_W, edge_b,
           en_W1, en_b1, en_W2, en_b2, gru_Wih, gru_Whh, gru_bih, gru_bhh):
    pad = E_PAD - E
    src = jnp.concatenate([edge_index[0], jnp.zeros((pad,), jnp.int32)])
    dst = jnp.concatenate(
        [edge_index[1], (jnp.arange(pad, dtype=jnp.int32) % N)])
    src = src.reshape(NCHUNK_P, CH)
    dst = dst.reshape(NCHUNK_P, CH)

    nfe = x_edge.shape[1]
    x8e = jnp.concatenate(
        [x_edge.reshape(E // PK, PK * nfe),
         jnp.zeros((EPK - E // PK, PK * nfe), _f32)])
    nfn = x_node.shape[1]
    x8n = jnp.concatenate(
        [x_node.reshape(NPR, PK * nfn),
         jnp.zeros((NPP - NPR, PK * nfn), _f32)])

    t8 = lambda b: jnp.tile(b.reshape(1, H), (1, PK))
    node_W8 = _kron8(node_W.T)
    node_b8 = t8(node_b)
    edge_W8 = _kron8(edge_W.T)
    edge_b8 = t8(edge_b)
    en_W18 = _kron8(en_W1.T)
    en_b18 = t8(en_b1)
    b28 = _kron8(en_b2.reshape(H, H))
    en_W2b = en_W2.astype(jnp.bfloat16)

    mats = tuple(_kron8(m) for m in (
        gru_Wih[0:H].T, gru_Wih[H:2 * H].T, gru_Wih[2 * H:3 * H].T,
        gru_Whh[0:H].T, gru_Whh[H:2 * H].T, gru_Whh[2 * H:3 * H].T))
    biases = (t8(gru_bih[0:H] + gru_bhh[0:H]),
              t8(gru_bih[H:2 * H] + gru_bhh[H:2 * H]),
              t8(gru_bih[2 * H:3 * H]),
              t8(gru_bhh[2 * H:3 * H]))

    ones_p = jnp.concatenate(
        [jnp.ones((E // PK, LW), _f32), jnp.zeros((EPK - E // PK, LW), _f32)])

    h = _embed_node(x8n, node_W8, node_b8)
    ef = _edge_feat(x8e, edge_W8, edge_b8, en_W18, en_b18)

    degp = _sc_scatter(ones_p, dst)

    for _ in range(N_LAYERS):
        hs = _sc_gather(h.reshape(NPP * PK, H), src)
        msg = _msg(hs, ef, en_W2b, b28)
        mp = _sc_scatter(msg, dst)
        h = _gru(mp, degp, h, mats, biases)
    return h.reshape(NPP * PK, H)[:N]


# speedup vs baseline: 1.1700x; 1.0108x over previous
import functools

import numpy as np
import jax
import jax.numpy as jnp
from jax import lax
from jax.experimental import pallas as pl
from jax.experimental.pallas import tpu as pltpu
from jax.experimental.pallas import tpu_sc as plsc

N = 10000
E = 160000
H = 32
HH = H * H
N_LAYERS = 3

PK = 8
LW = PK * H

NC, NS = 2, 16
NW = NC * NS
CH = 128
CPW = 40
NB = 8
NGROUPS = CPW // NB
NCHUNK_P = NW * CPW
E_PAD = NCHUNK_P * CH
EPK = E_PAD // PK

NPR = N // PK
NPP = 1280
SPR = 128
NPOUT = NS * SPR
ACC_ROWS = NPOUT * PK

BE = 1280
BEP = BE // PK
NBLK_REAL = E // BE
BNP = 256

_f32 = jnp.float32


def _dot(a, b):
    return jnp.dot(a, b, preferred_element_type=_f32)


def _embed_node_body(x_ref, w_ref, b_ref, o_ref):
    o_ref[...] = _dot(x_ref[...], w_ref[...]) + b_ref[...]


def _embed_node(x8, w8, b8):
    grid = NPP // BNP
    return pl.pallas_call(
        _embed_node_body,
        grid=(grid,),
        in_specs=[
            pl.BlockSpec((BNP, x8.shape[1]), lambda i: (i, 0)),
            pl.BlockSpec(w8.shape, lambda i: (0, 0)),
            pl.BlockSpec(b8.shape, lambda i: (0, 0)),
        ],
        out_specs=pl.BlockSpec((BNP, LW), lambda i: (i, 0)),
        out_shape=jax.ShapeDtypeStruct((NPP, LW), _f32),
    )(x8, w8, b8)


def _edge_feat_body(x_ref, w_ref, b_ref, w1_ref, b1_ref, o_ref):
    t = _dot(x_ref[...], w_ref[...]) + b_ref[...]
    o_ref[...] = jnp.maximum(_dot(t, w1_ref[...]) + b1_ref[...], 0.0)


def _edge_feat(x8, w8, b8, w18, b18):
    grid = E_PAD // BE
    return pl.pallas_call(
        _edge_feat_body,
        grid=(grid,),
        in_specs=[
            pl.BlockSpec((BEP, x8.shape[1]),
                         lambda i: (jnp.minimum(i, NBLK_REAL - 1), 0)),
            pl.BlockSpec(w8.shape, lambda i: (0, 0)),
            pl.BlockSpec(b8.shape, lambda i: (0, 0)),
            pl.BlockSpec(w18.shape, lambda i: (0, 0)),
            pl.BlockSpec(b18.shape, lambda i: (0, 0)),
        ],
        out_specs=pl.BlockSpec((BEP, LW), lambda i: (i, 0)),
        out_shape=jax.ShapeDtypeStruct((EPK, LW), _f32),
    )(x8, w8, b8, w18, b18)


def _msg_body(nblk_real, hs_ref, ef_ref, w2_ref, b28_ref, o_ref):
    @pl.when(pl.program_id(0) >= nblk_real)
    def _():
        o_ref[...] = jnp.zeros_like(o_ref)

    @pl.when(pl.program_id(0) < nblk_real)
    def _():
        hs = hs_ref[...]
        hsT = hs.T
        efT = ef_ref[...].T.astype(jnp.bfloat16)
        parts = []
        for q in range(PK):
            efq = efT[q * H:(q + 1) * H, :]
            gq = lax.dot_general(w2_ref[...], efq, (((1,), (0,)), ((), ())),
                                 preferred_element_type=_f32)
            hq = hsT[q * H:(q + 1) * H, :]
            acc = gq[0:H, :] * hq[0:1, :]
            for i in range(1, H):
                acc += gq[i * H:(i + 1) * H, :] * hq[i:i + 1, :]
            parts.append(acc)
        o_ref[...] = jnp.concatenate(parts, axis=0).T + _dot(hs, b28_ref[...])


def _msg(hs, ef, w2, b28, blk_off, nblk_real, grid):
    blk = pl.BlockSpec((BEP, LW), lambda i: (i, 0))
    efblk = pl.BlockSpec((BEP, LW), lambda i: (i + blk_off, 0))
    return pl.pallas_call(
        functools.partial(_msg_body, nblk_real),
        grid=(grid,),
        in_specs=[
            blk,
            efblk,
            pl.BlockSpec(w2.shape, lambda i: (0, 0)),
            pl.BlockSpec(b28.shape, lambda i: (0, 0)),
        ],
        out_specs=blk,
        out_shape=jax.ShapeDtypeStruct((grid * BEP, LW), _f32),
    )(hs, ef, w2, b28)


def _gru_body(pa0_ref, pa1_ref, pb0_ref, pb1_ref, d0_ref, d1_ref, h_ref,
              wir_ref, wiz_ref, win_ref, whr_ref, whz_ref, whn_ref,
              br_ref, bz_ref, bin_ref, bhn_ref, o_ref):
    deg = jnp.maximum(d0_ref[0] + d1_ref[0], 1.0)
    m = jnp.maximum(
        (pa0_ref[0] + pa1_ref[0] + pb0_ref[0] + pb1_ref[0]) / deg, 0.0)
    h = h_ref[...]
    r = jax.nn.sigmoid(_dot(m, wir_ref[...]) + _dot(h, whr_ref[...]) + br_ref[...])
    z = jax.nn.sigmoid(_dot(m, wiz_ref[...]) + _dot(h, whz_ref[...]) + bz_ref[...])
    n = jnp.tanh(_dot(m, win_ref[...]) + bin_ref[...]
                 + r * (_dot(h, whn_ref[...]) + bhn_ref[...]))
    o_ref[...] = (1.0 - z) * n + z * h


def _gru(mpa, mpb, degp, h, mats, biases):
    grid = NPP // BNP
    blk = pl.BlockSpec((BNP, LW), lambda i: (i, 0))
    p0 = pl.BlockSpec((1, BNP, LW), lambda i: (0, i, 0))
    p1 = pl.BlockSpec((1, BNP, LW), lambda i: (1, i, 0))
    full = lambda a: pl.BlockSpec(a.shape, lambda i: (0, 0))
    return pl.pallas_call(
        _gru_body,
        grid=(grid,),
        in_specs=[p0, p1, p0, p1, p0, p1, blk]
        + [full(m) for m in mats] + [full(b) for b in biases],
        out_specs=blk,
        out_shape=jax.ShapeDtypeStruct((NPP, LW), _f32),
    )(mpa, mpa, mpb, mpb, degp, degp, h, *mats, *biases)


@functools.cache
def _build_sc_kernels(chunk_lo, nchunks):
    mesh = plsc.VectorSubcoreMesh(core_axis_name="c", subcore_axis_name="s",
                                  num_cores=NC, num_subcores=NS)
    dma_sems = [pltpu.SemaphoreType.DMA] * (2 * NB)
    cpw = nchunks // NW
    ngroups = cpw // NB

    @functools.partial(
        pl.kernel, mesh=mesh,
        out_type=jax.ShapeDtypeStruct((nchunks, CH, H), _f32),
        scratch_types=[
            pltpu.VMEM((cpw, CH), jnp.int32),
            pltpu.VMEM((NB, CH, H), _f32),
        ] + dma_sems,
        compiler_params=pltpu.CompilerParams(use_tc_tiling_on_sc=False),
    )
    def sc_gather(h_hbm, src_hbm, out_hbm, idxall, rows, *sems):
        sg, sw = sems[:NB], sems[NB:]
        wid = lax.axis_index("s") * NC + lax.axis_index("c")
        base = wid * cpw

        pltpu.sync_copy(src_hbm.at[pl.ds(chunk_lo + base, cpw)], idxall)

        def start_g(j, b):
            pltpu.async_copy(h_hbm.at[idxall.at[j]], rows.at[b], sg[b])

        def wait_g(b):
            pltpu.make_async_copy(h_hbm.at[idxall.at[0]], rows.at[b],
                                  sg[b]).wait()

        def start_w(j, b):
            pltpu.async_copy(rows.at[b], out_hbm.at[base + j], sw[b])

        def wait_w(b):
            pltpu.make_async_copy(rows.at[b], out_hbm.at[0], sw[b]).wait()

        for b in range(NB):
            start_g(b, b)

        def body(g, carry):
            for b in range(NB):
                j = g * NB + b
                wait_g(b)
                start_w(j, b)
                wait_w(b)
                start_g(j + NB, b)
            return carry

        lax.fori_loop(0, ngroups - 1, body, 0)
        for b in range(NB):
            j = (ngroups - 1) * NB + b
            wait_g(b)
            start_w(j, b)
        for b in range(NB):
            wait_w(b)

    @functools.partial(
        pl.kernel, mesh=mesh,
        out_type=jax.ShapeDtypeStruct((NC, NPOUT * PK, H), _f32),
        scratch_types=[
            pltpu.VMEM((cpw, CH), jnp.int32),
            pltpu.VMEM((NB, CH, H), _f32),
            pltpu.VMEM((SPR * PK, H), _f32),
            pltpu.VMEM_SHARED((ACC_ROWS, H), _f32),
        ] + dma_sems,
        compiler_params=pltpu.CompilerParams(use_tc_tiling_on_sc=False),
    )
    def sc_scatter(vals_hbm, dst_hbm, out_hbm, idxall, vbuf, buf_v, acc,
                   *sems):
        sv, ss = sems[:NB], sems[NB:]
        c = lax.axis_index("c")
        s = lax.axis_index("s")
        wid = s * NC + c
        base = wid * cpw

        pltpu.sync_copy(dst_hbm.at[pl.ds(chunk_lo + base, cpw)], idxall)

        z = jnp.zeros((16,), _f32)

        def zbody(r, carry):
            buf_v[r, pl.ds(0, 16)] = z
            buf_v[r, pl.ds(16, 16)] = z
            return carry

        lax.fori_loop(0, SPR * PK, zbody, 0)
        pltpu.sync_copy(buf_v, acc.at[pl.ds(s * SPR * PK, SPR * PK)])
        plsc.subcore_barrier()

        def start_v(j, b):
            pltpu.async_copy(vals_hbm.at[base + j], vbuf.at[b], sv[b])

        def wait_v(b):
            pltpu.make_async_copy(vals_hbm.at[0], vbuf.at[b], sv[b]).wait()

        def start_s(j, b):
            pltpu.async_copy(vbuf.at[b], acc.at[idxall.at[j]], ss[b],
                             add=True)

        def wait_s(b):
            pltpu.make_async_copy(vbuf.at[b], acc.at[idxall.at[0]],
                                  ss[b]).wait()

        for b in range(NB):
            start_v(b, b)

        def body(g, carry):
            for b in range(NB):
                j = g * NB + b
                wait_v(b)
                start_s(j, b)
                wait_s(b)
                start_v(j + NB, b)
            return carry

        lax.fori_loop(0, ngroups - 1, body, 0)
        for b in range(NB):
            j = (ngroups - 1) * NB + b
            wait_v(b)
            start_s(j, b)
            wait_s(b)

        plsc.subcore_barrier()
        pltpu.sync_copy(acc.at[pl.ds(s * SPR * PK, SPR * PK)], buf_v)
        pltpu.sync_copy(buf_v, out_hbm.at[c].at[pl.ds(s * SPR * PK, SPR * PK)])

    return sc_gather, sc_scatter


def _sc_gather(h32, src2d, chunk_lo, nchunks):
    out = _build_sc_kernels(chunk_lo, nchunks)[0](h32, src2d)
    return out.reshape(nchunks * CH // PK, LW)


def _sc_scatter(vals_p, dst2d, chunk_lo, nchunks):
    out = _build_sc_kernels(chunk_lo, nchunks)[1](
        vals_p.reshape(nchunks, CH, H), dst2d)
    return out.reshape(NC, NPOUT, LW)


def _kron8(w):
    return jnp.kron(jnp.eye(PK, dtype=_f32), w)


def kernel(x_node, x_edge, edge_index, node_W, node_b, edge_W, edge_b,
           en_W1, en_b1, en_W2, en_b2, gru_Wih, gru_Whh, gru_bih, gru_bhh):
    pad = E_PAD - E
    src = jnp.concatenate([edge_index[0], jnp.zeros((pad,), jnp.int32)])
    dst = jnp.concatenate(
        [edge_index[1], (jnp.arange(pad, dtype=jnp.int32) % N)])
    src = src.reshape(NCHUNK_P, CH)
    dst = dst.reshape(NCHUNK_P, CH)

    nfe = x_edge.shape[1]
    x8e = jnp.concatenate(
        [x_edge.reshape(E // PK, PK * nfe),
         jnp.zeros((EPK - E // PK, PK * nfe), _f32)])
    nfn = x_node.shape[1]
    x8n = jnp.concatenate(
        [x_node.reshape(NPR, PK * nfn),
         jnp.zeros((NPP - NPR, PK * nfn), _f32)])

    t8 = lambda b: jnp.tile(b.reshape(1, H), (1, PK))
    node_W8 = _kron8(node_W.T)
    node_b8 = t8(node_b)
    edge_W8 = _kron8(edge_W.T)
    edge_b8 = t8(edge_b)
    en_W18 = _kron8(en_W1.T)
    en_b18 = t8(en_b1)
    b28 = _kron8(en_b2.reshape(H, H))
    en_W2b = en_W2.astype(jnp.bfloat16)

    mats = tuple(_kron8(m) for m in (
        gru_Wih[0:H].T, gru_Wih[H:2 * H].T, gru_Wih[2 * H:3 * H].T,
        gru_Whh[0:H].T, gru_Whh[H:2 * H].T, gru_Whh[2 * H:3 * H].T))
    biases = (t8(gru_bih[0:H] + gru_bhh[0:H]),
              t8(gru_bih[H:2 * H] + gru_bhh[H:2 * H]),
              t8(gru_bih[2 * H:3 * H]),
              t8(gru_bhh[2 * H:3 * H]))

    ones_p = jnp.concatenate(
        [jnp.ones((E // PK, LW), _f32), jnp.zeros((EPK - E // PK, LW), _f32)])

    h = _embed_node(x8n, node_W8, node_b8)
    ef = _edge_feat(x8e, edge_W8, edge_b8, en_W18, en_b18)

    degp = _sc_scatter(ones_p, dst, 0, NCHUNK_P)

    HC = NCHUNK_P // 2
    HB = E_PAD // BE // 2
    NBLK_B = NBLK_REAL - HB
    for _ in range(N_LAYERS):
        h32 = h.reshape(NPP * PK, H)
        hsa = _sc_gather(h32, src, 0, HC)
        hsb = _sc_gather(h32, src, HC, HC)
        msga = _msg(hsa, ef, en_W2b, b28, 0, HB, HB)
        msgb = _msg(hsb, ef, en_W2b, b28, HB, NBLK_B, HB)
        mpa = _sc_scatter(msga, dst, 0, HC)
        mpb = _sc_scatter(msgb, dst, HC, HC)
        h = _gru(mpa, mpb, degp, h, mats, biases)
    return h.reshape(NPP * PK, H)[:N]
